# baseline (speedup 1.0000x reference)
import functools

import jax
import jax.numpy as jnp
from jax import lax
from jax.experimental import pallas as pl
from jax.experimental.pallas import tpu as pltpu
from jax.experimental.pallas import tpu_sc as plsc

B, DIM, H, W = 8, 768, 32, 32
E, K = 8, 2
HIDDEN = DIM * 2
N_TOK = B * H * W
TB = 1024
NB = N_TOK // TB
NPAIR = K * N_TOK
BLK = 512
T_TILES = NPAIR // BLK + E
NROWS = T_TILES * BLK

NC, NS = 2, 16
NWK = NC * NS
CH = 64
CHC = 32

_INV_SQRT2 = 0.7071067811865476


def _router_body(x_ref, rw_ref, rb_ref,
                 tok_ref, topi_ref, rank_ref, w0_ref, w1_ref, counts_ref,
                 carry_ref, u_ref, ident_ref):
    b = pl.program_id(0)

    @pl.when(b == 0)
    def _():
        carry_ref[...] = jnp.zeros_like(carry_ref)
        ii2 = lax.broadcasted_iota(jnp.int32, (2 * TB, 2 * TB), 0)
        jj2 = lax.broadcasted_iota(jnp.int32, (2 * TB, 2 * TB), 1)
        u_ref[...] = (ii2 < jj2).astype(jnp.float32)
        ii1 = lax.broadcasted_iota(jnp.int32, (TB, TB), 0)
        jj1 = lax.broadcasted_iota(jnp.int32, (TB, TB), 1)
        ident_ref[...] = (ii1 == jj1).astype(jnp.float32)

    xb = jnp.transpose(x_ref[0], (1, 0))
    tok_ref[...] = xb
    logits = lax.dot_general(xb, rw_ref[...],
                             (((1,), (1,)), ((), ())),
                             preferred_element_type=jnp.float32)
    logits = logits + rb_ref[...]
    idx = lax.broadcasted_iota(jnp.int32, (TB, E), 1)
    m1 = jnp.max(logits, axis=1, keepdims=True)
    i1 = jnp.min(jnp.where(logits == m1, idx, E), axis=1, keepdims=True)
    l2 = jnp.where(idx == i1, -jnp.inf, logits)
    m2 = jnp.max(l2, axis=1, keepdims=True)
    i2 = jnp.min(jnp.where(l2 == m2, idx, E), axis=1, keepdims=True)
    e21 = jnp.exp(m2 - m1)
    w1 = 1.0 / (1.0 + e21)
    w2 = 1.0 - w1
    w0_ref[...] = w1
    w1_ref[...] = w2

    ident = ident_ref[...]
    i1r = lax.dot_general(i1.astype(jnp.float32), ident,
                          (((0,), (0,)), ((), ())),
                          preferred_element_type=jnp.float32)
    i2r = lax.dot_general(i2.astype(jnp.float32), ident,
                          (((0,), (0,)), ((), ())),
                          preferred_element_type=jnp.float32)
    ir = jnp.concatenate([i1r, i2r], axis=1).astype(jnp.int32)

    idx_e = lax.broadcasted_iota(jnp.int32, (E, 2 * TB), 0)
    mt = (idx_e == ir).astype(jnp.float32)
    prefix = lax.dot_general(mt, u_ref[...],
                             (((1,), (0,)), ((), ())),
                             preferred_element_type=jnp.float32)
    prefix = prefix + carry_ref[...]
    rank_row = jnp.sum(mt * prefix, axis=0, keepdims=True)
    carry_ref[...] += jnp.sum(mt, axis=1, keepdims=True)
    counts_ref[...] = carry_ref[...]

    topi_ref[...] = ir.reshape(1, 1, 2 * TB)
    rank_ref[...] = rank_row.astype(jnp.int32).reshape(1, 1, 2 * TB)


_router = pl.pallas_call(
    _router_body,
    grid=(NB,),
    in_specs=[
        pl.BlockSpec((1, DIM, TB), lambda b: (b, 0, 0)),
        pl.BlockSpec((E, DIM), lambda b: (0, 0)),
        pl.BlockSpec((1, E), lambda b: (0, 0)),
    ],
    out_specs=[
        pl.BlockSpec((TB, DIM), lambda b: (b, 0)),
        pl.BlockSpec((1, 1, 2 * TB), lambda b: (b, 0, 0)),
        pl.BlockSpec((1, 1, 2 * TB), lambda b: (b, 0, 0)),
        pl.BlockSpec((TB, 1), lambda b: (b, 0)),
        pl.BlockSpec((TB, 1), lambda b: (b, 0)),
        pl.BlockSpec((E, 1), lambda b: (0, 0)),
    ],
    out_shape=[
        jax.ShapeDtypeStruct((N_TOK, DIM), jnp.float32),
        jax.ShapeDtypeStruct((NB, 1, 2 * TB), jnp.int32),
        jax.ShapeDtypeStruct((NB, 1, 2 * TB), jnp.int32),
        jax.ShapeDtypeStruct((N_TOK, 1), jnp.float32),
        jax.ShapeDtypeStruct((N_TOK, 1), jnp.float32),
        jax.ShapeDtypeStruct((E, 1), jnp.float32),
    ],
    scratch_shapes=[
        pltpu.VMEM((E, 1), jnp.float32),
        pltpu.VMEM((2 * TB, 2 * TB), jnp.float32),
        pltpu.VMEM((TB, TB), jnp.float32),
    ],
    compiler_params=pltpu.CompilerParams(dimension_semantics=("arbitrary",)),
)


def _posmap_body(start_ref, topi_ref, rank_ref, pos_ref):
    t = topi_ref[...]
    acc = rank_ref[...]
    for e in range(E):
        acc = acc + jnp.where(t == e, start_ref[e], 0)
    pos_ref[...] = acc


_posmap = pl.pallas_call(
    _posmap_body,
    grid_spec=pltpu.PrefetchScalarGridSpec(
        num_scalar_prefetch=1,
        grid=(NB,),
        in_specs=[
            pl.BlockSpec((1, 1, 2 * TB), lambda b, s: (b, 0, 0)),
            pl.BlockSpec((1, 1, 2 * TB), lambda b, s: (b, 0, 0)),
        ],
        out_specs=pl.BlockSpec((1, 1, 2 * TB), lambda b, s: (b, 0, 0)),
    ),
    out_shape=jax.ShapeDtypeStruct((NB, 1, 2 * TB), jnp.int32),
    compiler_params=pltpu.CompilerParams(dimension_semantics=("arbitrary",)),
)


_sc_mesh = plsc.VectorSubcoreMesh(core_axis_name="c", subcore_axis_name="s")

_PW = NPAIR // NWK
_NCH = _PW // CH


@functools.partial(
    pl.kernel,
    mesh=_sc_mesh,
    out_type=jax.ShapeDtypeStruct((NROWS, DIM), jnp.float32),
    scratch_types=[
        pltpu.VMEM((_PW,), jnp.int32),
        pltpu.VMEM((CH,), jnp.int32),
        pltpu.VMEM((CH,), jnp.int32),
        pltpu.VMEM((CH,), jnp.int32),
        pltpu.VMEM((CH,), jnp.int32),
        pltpu.VMEM((CH, DIM), jnp.float32),
        pltpu.VMEM((CH, DIM), jnp.float32),
        pltpu.SemaphoreType.DMA,
        pltpu.SemaphoreType.DMA,
        pltpu.SemaphoreType.DMA,
        pltpu.SemaphoreType.DMA,
    ],
)
def _dispatch(tok_hbm, pos_hbm, xs_hbm,
              pos_all, pos_c0, pos_c1, tok_c0, tok_c1, row0, row1,
              g0, g1, s0, s1):
    wid = lax.axis_index("s") * NC + lax.axis_index("c")
    base = wid * _PW
    pos_c = [pos_c0, pos_c1]
    tok_c = [tok_c0, tok_c1]
    row = [row0, row1]
    gsem = [g0, g1]
    ssem = [s0, s1]
    pltpu.sync_copy(pos_hbm.at[pl.ds(base, _PW)], pos_all)

    def fill(bi, c):
        for j in range(CH // 16):
            sl = pl.ds(16 * j, 16)
            pos_c[bi][sl] = pos_all[pl.ds(c * CH + 16 * j, 16)]
            pvec = jnp.full((16,), base + c * CH + 16 * j, jnp.int32) + \
                lax.iota(jnp.int32, 16)
            tok_c[bi][sl] = ((pvec >> 11) << 10) + (pvec & (TB - 1))

    d_g = [None, None]
    d_s = [None, None]
    fill(0, 0)
    d_g[0] = pltpu.async_copy(tok_hbm.at[tok_c[0]], row[0], gsem[0])
    for c in range(_NCH):
        cb = c % 2
        nb = (c + 1) % 2
        if c + 1 < _NCH:
            if d_s[nb] is not None:
                d_s[nb].wait()
            fill(nb, c + 1)
            d_g[nb] = pltpu.async_copy(tok_hbm.at[tok_c[nb]], row[nb],
                                       gsem[nb])
        d_g[cb].wait()
        d_s[cb] = pltpu.async_copy(row[cb], xs_hbm.at[pos_c[cb]], ssem[cb])
    d_s[0].wait()
    d_s[1].wait()


def _gmm_body(te_ref, xs_ref, f1w_ref, f1b_ref, f2w_ref, f2b_ref, out_ref):
    @pl.when(pl.program_id(0) < te_ref[T_TILES])
    def _():
        xb = xs_ref[...].astype(jnp.bfloat16)
        h = lax.dot_general(xb, f1w_ref[0],
                            (((1,), (1,)), ((), ())),
                            preferred_element_type=jnp.float32)
        h = h + f1b_ref[0]
        h = 0.5 * h * (1.0 + lax.erf(h * _INV_SQRT2))
        y = lax.dot_general(h.astype(jnp.bfloat16), f2w_ref[0],
                            (((1,), (1,)), ((), ())),
                            preferred_element_type=jnp.float32)
        out_ref[...] = y + f2b_ref[0]


_gmm = pl.pallas_call(
    _gmm_body,
    grid_spec=pltpu.PrefetchScalarGridSpec(
        num_scalar_prefetch=1,
        grid=(T_TILES,),
        in_specs=[
            pl.BlockSpec((BLK, DIM), lambda t, te: (t, 0)),
            pl.BlockSpec((1, HIDDEN, DIM), lambda t, te: (te[t], 0, 0)),
            pl.BlockSpec((1, 1, HIDDEN), lambda t, te: (te[t], 0, 0)),
            pl.BlockSpec((1, DIM, HIDDEN), lambda t, te: (te[t], 0, 0)),
            pl.BlockSpec((1, 1, DIM), lambda t, te: (te[t], 0, 0)),
        ],
        out_specs=pl.BlockSpec((BLK, DIM), lambda t, te: (t, 0)),
    ),
    out_shape=jax.ShapeDtypeStruct((NROWS, DIM), jnp.float32),
    compiler_params=pltpu.CompilerParams(dimension_semantics=("arbitrary",)),
)


_TW = N_TOK // NWK
_NCHC = _TW // CHC


@functools.partial(
    pl.kernel,
    mesh=_sc_mesh,
    out_type=(
        jax.ShapeDtypeStruct((N_TOK, DIM), jnp.float32),
        jax.ShapeDtypeStruct((N_TOK, DIM), jnp.float32),
    ),
    scratch_types=[
        pltpu.VMEM((_TW,), jnp.int32),
        pltpu.VMEM((_TW,), jnp.int32),
        pltpu.VMEM((CHC,), jnp.int32),
        pltpu.VMEM((CHC,), jnp.int32),
        pltpu.VMEM((CHC,), jnp.int32),
        pltpu.VMEM((CHC,), jnp.int32),
        pltpu.VMEM((CHC, DIM), jnp.float32),
        pltpu.VMEM((CHC, DIM), jnp.float32),
        pltpu.VMEM((CHC, DIM), jnp.float32),
        pltpu.VMEM((CHC, DIM), jnp.float32),
        pltpu.SemaphoreType.DMA,
        pltpu.SemaphoreType.DMA,
        pltpu.SemaphoreType.DMA,
        pltpu.SemaphoreType.DMA,
        pltpu.SemaphoreType.DMA,
        pltpu.SemaphoreType.DMA,
        pltpu.SemaphoreType.DMA,
        pltpu.SemaphoreType.DMA,
    ],
)
def _combine(yw_hbm, pos_hbm, yg0_hbm, yg1_hbm,
             p0_all, p1_all, i00, i01, i10, i11,
             b00, b01, b10, b11,
             g00, g01, g10, g11, s00, s01, s10, s11):
    wid = lax.axis_index("s") * NC + lax.axis_index("c")
    tbase = wid * _TW
    p0base = (tbase // TB) * (2 * TB) + (tbase % TB)
    p1base = p0base + TB
    pltpu.sync_copy(pos_hbm.at[pl.ds(p0base, _TW)], p0_all)
    pltpu.sync_copy(pos_hbm.at[pl.ds(p1base, _TW)], p1_all)

    p_all = [p0_all, p1_all]
    idx = [[i00, i01], [i10, i11]]
    buf = [[b00, b01], [b10, b11]]
    gsem = [[g00, g01], [g10, g11]]
    ssem = [[s00, s01], [s10, s11]]
    out = [yg0_hbm, yg1_hbm]

    def fill(k, bi, c):
        for j in range(CHC // 16):
            sl = pl.ds(16 * j, 16)
            idx[k][bi][sl] = p_all[k][pl.ds(c * CHC + 16 * j, 16)]

    d_g = [[None, None], [None, None]]
    d_s = [[None, None], [None, None]]
    for k in (0, 1):
        fill(k, 0, 0)
        d_g[k][0] = pltpu.async_copy(yw_hbm.at[idx[k][0]], buf[k][0],
                                     gsem[k][0])
    for c in range(_NCHC):
        cb = c % 2
        nb = (c + 1) % 2
        for k in (0, 1):
            if c + 1 < _NCHC:
                if d_s[k][nb] is not None:
                    d_s[k][nb].wait()
                fill(k, nb, c + 1)
                d_g[k][nb] = pltpu.async_copy(yw_hbm.at[idx[k][nb]],
                                              buf[k][nb], gsem[k][nb])
            d_g[k][cb].wait()
            d_s[k][cb] = pltpu.async_copy(
                buf[k][cb],
                out[k].at[pl.ds(tbase + c * CHC, CHC), :], ssem[k][cb])
    for k in (0, 1):
        d_s[k][0].wait()
        d_s[k][1].wait()


def _finish_body(x_ref, a_ref, b_ref, w0_ref, w1_ref, scale_ref, out_ref):
    s = scale_ref[0, 0]
    res = x_ref[...] + s * (w0_ref[...] * a_ref[...] +
                            w1_ref[...] * b_ref[...])
    out_ref[...] = jnp.transpose(res, (1, 0)).reshape(1, DIM, TB)


_finish = pl.pallas_call(
    _finish_body,
    grid=(NB,),
    in_specs=[
        pl.BlockSpec((TB, DIM), lambda b: (b, 0)),
        pl.BlockSpec((TB, DIM), lambda b: (b, 0)),
        pl.BlockSpec((TB, DIM), lambda b: (b, 0)),
        pl.BlockSpec((TB, 1), lambda b: (b, 0)),
        pl.BlockSpec((TB, 1), lambda b: (b, 0)),
        pl.BlockSpec((1, 1), lambda b: (0, 0)),
    ],
    out_specs=pl.BlockSpec((1, DIM, TB), lambda b: (b, 0, 0)),
    out_shape=jax.ShapeDtypeStruct((B, DIM, TB), jnp.float32),
    compiler_params=pltpu.CompilerParams(dimension_semantics=("arbitrary",)),
)


@jax.jit
def _moe(x4, router_w, router_b, f1w, f1b, f2w, f2b, scale):
    tokens, topi3, rank3, w0, w1, counts_f = _router(
        x4, router_w, router_b.reshape(1, E))

    counts = counts_f[:, 0].astype(jnp.int32)
    nt = (counts + BLK - 1) // BLK
    tile_cum = jnp.cumsum(nt)
    start = (BLK * (tile_cum - nt)).astype(jnp.int32)
    t_idx = jnp.arange(T_TILES, dtype=jnp.int32)
    te_map = jnp.minimum(
        jnp.sum((t_idx[:, None] >= tile_cum[None, :]).astype(jnp.int32),
                axis=1), E - 1).astype(jnp.int32)
    te_plus = jnp.concatenate([te_map, tile_cum[-1:]])

    pos3 = _posmap(start, topi3, rank3)
    pos = pos3.reshape(NPAIR)
    xs = _dispatch(tokens, pos)
    yw = _gmm(te_plus, xs,
              f1w.astype(jnp.bfloat16), f1b.reshape(E, 1, HIDDEN),
              f2w.astype(jnp.bfloat16), f2b.reshape(E, 1, DIM))
    yg0, yg1 = _combine(yw, pos)
    return _finish(tokens, yg0, yg1, w0, w1, scale.reshape(1, 1))


def kernel(x, router_w, router_b, fc1_w, fc1_b, fc2_w, fc2_b, scale):
    b, c, h, w = x.shape
    out = _moe(x.reshape(b, c, h * w), router_w, router_b,
               fc1_w, fc1_b, fc2_w, fc2_b, scale)
    return out.reshape(b, c, h, w)


# speedup vs baseline: 1.0740x; 1.0740x over previous
import functools

import jax
import jax.numpy as jnp
from jax import lax
from jax.experimental import pallas as pl
from jax.experimental.pallas import tpu as pltpu
from jax.experimental.pallas import tpu_sc as plsc

B, DIM, H, W = 8, 768, 32, 32
E, K = 8, 2
HIDDEN = DIM * 2
N_TOK = B * H * W
TB = 1024
NB = N_TOK // TB
NPAIR = K * N_TOK
BLK = 512
T_TILES = NPAIR // BLK + E
NROWS = T_TILES * BLK

NC, NS = 2, 16
NWK = NC * NS
CH = 64
CHC = 32

_INV_SQRT2 = 0.7071067811865476


def _router_body(tok_ref, rw_ref, rb_ref,
                 topi_ref, rank_ref, w0_ref, w1_ref, counts_ref,
                 carry_ref, u_ref, ident_ref):
    b = pl.program_id(0)

    @pl.when(b == 0)
    def _():
        carry_ref[...] = jnp.zeros_like(carry_ref)
        ii2 = lax.broadcasted_iota(jnp.int32, (2 * TB, 2 * TB), 0)
        jj2 = lax.broadcasted_iota(jnp.int32, (2 * TB, 2 * TB), 1)
        u_ref[...] = (ii2 < jj2).astype(jnp.bfloat16)
        ii1 = lax.broadcasted_iota(jnp.int32, (TB, TB), 0)
        jj1 = lax.broadcasted_iota(jnp.int32, (TB, TB), 1)
        ident_ref[...] = (ii1 == jj1).astype(jnp.float32)

    xb = tok_ref[...]
    logits = lax.dot_general(xb, rw_ref[...],
                             (((1,), (1,)), ((), ())),
                             preferred_element_type=jnp.float32)
    logits = logits + rb_ref[...]
    idx = lax.broadcasted_iota(jnp.int32, (TB, E), 1)
    m1 = jnp.max(logits, axis=1, keepdims=True)
    i1 = jnp.min(jnp.where(logits == m1, idx, E), axis=1, keepdims=True)
    l2 = jnp.where(idx == i1, -jnp.inf, logits)
    m2 = jnp.max(l2, axis=1, keepdims=True)
    i2 = jnp.min(jnp.where(l2 == m2, idx, E), axis=1, keepdims=True)
    e21 = jnp.exp(m2 - m1)
    w1 = 1.0 / (1.0 + e21)
    w2 = 1.0 - w1
    w0_ref[...] = w1
    w1_ref[...] = w2

    ident = ident_ref[...]
    i1r = lax.dot_general(i1.astype(jnp.float32), ident,
                          (((0,), (0,)), ((), ())),
                          preferred_element_type=jnp.float32)
    i2r = lax.dot_general(i2.astype(jnp.float32), ident,
                          (((0,), (0,)), ((), ())),
                          preferred_element_type=jnp.float32)
    ir = jnp.concatenate([i1r, i2r], axis=1).astype(jnp.int32)

    idx_e = lax.broadcasted_iota(jnp.int32, (E, 2 * TB), 0)
    mt = (idx_e == ir).astype(jnp.float32)
    prefix = lax.dot_general(mt.astype(jnp.bfloat16), u_ref[...],
                             (((1,), (0,)), ((), ())),
                             preferred_element_type=jnp.float32)
    prefix = prefix + carry_ref[...]
    rank_row = jnp.sum(mt * prefix, axis=0, keepdims=True)
    carry_ref[...] += jnp.sum(mt, axis=1, keepdims=True)
    counts_ref[...] = carry_ref[...]

    topi_ref[...] = ir.reshape(1, 1, 2 * TB)
    rank_ref[...] = rank_row.astype(jnp.int32).reshape(1, 1, 2 * TB)


_router = pl.pallas_call(
    _router_body,
    grid=(NB,),
    in_specs=[
        pl.BlockSpec((TB, DIM), lambda b: (b, 0)),
        pl.BlockSpec((E, DIM), lambda b: (0, 0)),
        pl.BlockSpec((1, E), lambda b: (0, 0)),
    ],
    out_specs=[
        pl.BlockSpec((1, 1, 2 * TB), lambda b: (b, 0, 0)),
        pl.BlockSpec((1, 1, 2 * TB), lambda b: (b, 0, 0)),
        pl.BlockSpec((TB, 1), lambda b: (b, 0)),
        pl.BlockSpec((TB, 1), lambda b: (b, 0)),
        pl.BlockSpec((E, 1), lambda b: (0, 0)),
    ],
    out_shape=[
        jax.ShapeDtypeStruct((NB, 1, 2 * TB), jnp.int32),
        jax.ShapeDtypeStruct((NB, 1, 2 * TB), jnp.int32),
        jax.ShapeDtypeStruct((N_TOK, 1), jnp.float32),
        jax.ShapeDtypeStruct((N_TOK, 1), jnp.float32),
        jax.ShapeDtypeStruct((E, 1), jnp.float32),
    ],
    scratch_shapes=[
        pltpu.VMEM((E, 1), jnp.float32),
        pltpu.VMEM((2 * TB, 2 * TB), jnp.bfloat16),
        pltpu.VMEM((TB, TB), jnp.float32),
    ],
    compiler_params=pltpu.CompilerParams(dimension_semantics=("arbitrary",)),
)


def _posmap_body(start_ref, topi_ref, rank_ref, pos_ref):
    t = topi_ref[...]
    acc = rank_ref[...]
    for e in range(E):
        acc = acc + jnp.where(t == e, start_ref[e], 0)
    pos_ref[...] = acc


_posmap = pl.pallas_call(
    _posmap_body,
    grid_spec=pltpu.PrefetchScalarGridSpec(
        num_scalar_prefetch=1,
        grid=(NB,),
        in_specs=[
            pl.BlockSpec((1, 1, 2 * TB), lambda b, s: (b, 0, 0)),
            pl.BlockSpec((1, 1, 2 * TB), lambda b, s: (b, 0, 0)),
        ],
        out_specs=pl.BlockSpec((1, 1, 2 * TB), lambda b, s: (b, 0, 0)),
    ),
    out_shape=jax.ShapeDtypeStruct((NB, 1, 2 * TB), jnp.int32),
    compiler_params=pltpu.CompilerParams(dimension_semantics=("arbitrary",)),
)


_sc_mesh = plsc.VectorSubcoreMesh(core_axis_name="c", subcore_axis_name="s")

_PW = NPAIR // NWK
_NCH = _PW // CH


@functools.partial(
    pl.kernel,
    mesh=_sc_mesh,
    out_type=jax.ShapeDtypeStruct((NROWS, DIM), jnp.float32),
    scratch_types=[
        pltpu.VMEM((_PW,), jnp.int32),
        pltpu.VMEM((CH,), jnp.int32),
        pltpu.VMEM((CH,), jnp.int32),
        pltpu.VMEM((CH,), jnp.int32),
        pltpu.VMEM((CH,), jnp.int32),
        pltpu.VMEM((CH, DIM), jnp.float32),
        pltpu.VMEM((CH, DIM), jnp.float32),
        pltpu.SemaphoreType.DMA,
        pltpu.SemaphoreType.DMA,
        pltpu.SemaphoreType.DMA,
        pltpu.SemaphoreType.DMA,
    ],
)
def _dispatch(tok_hbm, pos_hbm, xs_hbm,
              pos_all, pos_c0, pos_c1, tok_c0, tok_c1, row0, row1,
              g0, g1, s0, s1):
    wid = lax.axis_index("s") * NC + lax.axis_index("c")
    base = wid * _PW
    pos_c = [pos_c0, pos_c1]
    tok_c = [tok_c0, tok_c1]
    row = [row0, row1]
    gsem = [g0, g1]
    ssem = [s0, s1]
    pltpu.sync_copy(pos_hbm.at[pl.ds(base, _PW)], pos_all)

    def fill(bi, c):
        for j in range(CH // 16):
            sl = pl.ds(16 * j, 16)
            pos_c[bi][sl] = pos_all[pl.ds(c * CH + 16 * j, 16)]
            pvec = jnp.full((16,), base + c * CH + 16 * j, jnp.int32) + \
                lax.iota(jnp.int32, 16)
            tok_c[bi][sl] = ((pvec >> 11) << 10) + (pvec & (TB - 1))

    d_g = [None, None]
    d_s = [None, None]
    fill(0, 0)
    d_g[0] = pltpu.async_copy(tok_hbm.at[tok_c[0]], row[0], gsem[0])
    for c in range(_NCH):
        cb = c % 2
        nb = (c + 1) % 2
        if c + 1 < _NCH:
            if d_s[nb] is not None:
                d_s[nb].wait()
            fill(nb, c + 1)
            d_g[nb] = pltpu.async_copy(tok_hbm.at[tok_c[nb]], row[nb],
                                       gsem[nb])
        d_g[cb].wait()
        d_s[cb] = pltpu.async_copy(row[cb], xs_hbm.at[pos_c[cb]], ssem[cb])
    d_s[0].wait()
    d_s[1].wait()


def _gmm_body(te_ref, xs_ref, f1w_ref, f1b_ref, f2w_ref, f2b_ref, out_ref):
    @pl.when(pl.program_id(0) < te_ref[T_TILES])
    def _():
        xb = xs_ref[...].astype(jnp.bfloat16)
        h = lax.dot_general(xb, f1w_ref[0],
                            (((1,), (1,)), ((), ())),
                            preferred_element_type=jnp.float32)
        h = h + f1b_ref[0]
        h = 0.5 * h * (1.0 + lax.erf(h * _INV_SQRT2))
        y = lax.dot_general(h.astype(jnp.bfloat16), f2w_ref[0],
                            (((1,), (1,)), ((), ())),
                            preferred_element_type=jnp.float32)
        out_ref[...] = y + f2b_ref[0]


_gmm = pl.pallas_call(
    _gmm_body,
    grid_spec=pltpu.PrefetchScalarGridSpec(
        num_scalar_prefetch=1,
        grid=(T_TILES,),
        in_specs=[
            pl.BlockSpec((BLK, DIM), lambda t, te: (t, 0)),
            pl.BlockSpec((1, HIDDEN, DIM), lambda t, te: (te[t], 0, 0)),
            pl.BlockSpec((1, 1, HIDDEN), lambda t, te: (te[t], 0, 0)),
            pl.BlockSpec((1, DIM, HIDDEN), lambda t, te: (te[t], 0, 0)),
            pl.BlockSpec((1, 1, DIM), lambda t, te: (te[t], 0, 0)),
        ],
        out_specs=pl.BlockSpec((BLK, DIM), lambda t, te: (t, 0)),
    ),
    out_shape=jax.ShapeDtypeStruct((NROWS, DIM), jnp.float32),
    compiler_params=pltpu.CompilerParams(dimension_semantics=("arbitrary",)),
)


_TW = N_TOK // NWK
_NCHC = _TW // CHC


@functools.partial(
    pl.kernel,
    mesh=_sc_mesh,
    out_type=(
        jax.ShapeDtypeStruct((N_TOK, DIM), jnp.float32),
        jax.ShapeDtypeStruct((N_TOK, DIM), jnp.float32),
    ),
    scratch_types=[
        pltpu.VMEM((_TW,), jnp.int32),
        pltpu.VMEM((_TW,), jnp.int32),
        pltpu.VMEM((CHC,), jnp.int32),
        pltpu.VMEM((CHC,), jnp.int32),
        pltpu.VMEM((CHC,), jnp.int32),
        pltpu.VMEM((CHC,), jnp.int32),
        pltpu.VMEM((CHC, DIM), jnp.float32),
        pltpu.VMEM((CHC, DIM), jnp.float32),
        pltpu.VMEM((CHC, DIM), jnp.float32),
        pltpu.VMEM((CHC, DIM), jnp.float32),
        pltpu.SemaphoreType.DMA,
        pltpu.SemaphoreType.DMA,
        pltpu.SemaphoreType.DMA,
        pltpu.SemaphoreType.DMA,
        pltpu.SemaphoreType.DMA,
        pltpu.SemaphoreType.DMA,
        pltpu.SemaphoreType.DMA,
        pltpu.SemaphoreType.DMA,
    ],
)
def _combine(yw_hbm, pos_hbm, yg0_hbm, yg1_hbm,
             p0_all, p1_all, i00, i01, i10, i11,
             b00, b01, b10, b11,
             g00, g01, g10, g11, s00, s01, s10, s11):
    wid = lax.axis_index("s") * NC + lax.axis_index("c")
    tbase = wid * _TW
    p0base = (tbase // TB) * (2 * TB) + (tbase % TB)
    p1base = p0base + TB
    pltpu.sync_copy(pos_hbm.at[pl.ds(p0base, _TW)], p0_all)
    pltpu.sync_copy(pos_hbm.at[pl.ds(p1base, _TW)], p1_all)

    p_all = [p0_all, p1_all]
    idx = [[i00, i01], [i10, i11]]
    buf = [[b00, b01], [b10, b11]]
    gsem = [[g00, g01], [g10, g11]]
    ssem = [[s00, s01], [s10, s11]]
    out = [yg0_hbm, yg1_hbm]

    def fill(k, bi, c):
        for j in range(CHC // 16):
            sl = pl.ds(16 * j, 16)
            idx[k][bi][sl] = p_all[k][pl.ds(c * CHC + 16 * j, 16)]

    d_g = [[None, None], [None, None]]
    d_s = [[None, None], [None, None]]
    for k in (0, 1):
        fill(k, 0, 0)
        d_g[k][0] = pltpu.async_copy(yw_hbm.at[idx[k][0]], buf[k][0],
                                     gsem[k][0])
    for c in range(_NCHC):
        cb = c % 2
        nb = (c + 1) % 2
        for k in (0, 1):
            if c + 1 < _NCHC:
                if d_s[k][nb] is not None:
                    d_s[k][nb].wait()
                fill(k, nb, c + 1)
                d_g[k][nb] = pltpu.async_copy(yw_hbm.at[idx[k][nb]],
                                              buf[k][nb], gsem[k][nb])
            d_g[k][cb].wait()
            d_s[k][cb] = pltpu.async_copy(
                buf[k][cb],
                out[k].at[pl.ds(tbase + c * CHC, CHC), :], ssem[k][cb])
    for k in (0, 1):
        d_s[k][0].wait()
        d_s[k][1].wait()


def _finish_body(x_ref, a_ref, b_ref, w0_ref, w1_ref, scale_ref, out_ref):
    s = scale_ref[0, 0]
    res = x_ref[...] + s * (w0_ref[...] * a_ref[...] +
                            w1_ref[...] * b_ref[...])
    out_ref[...] = jnp.transpose(res, (1, 0)).reshape(1, DIM, TB)


_finish = pl.pallas_call(
    _finish_body,
    grid=(NB,),
    in_specs=[
        pl.BlockSpec((TB, DIM), lambda b: (b, 0)),
        pl.BlockSpec((TB, DIM), lambda b: (b, 0)),
        pl.BlockSpec((TB, DIM), lambda b: (b, 0)),
        pl.BlockSpec((TB, 1), lambda b: (b, 0)),
        pl.BlockSpec((TB, 1), lambda b: (b, 0)),
        pl.BlockSpec((1, 1), lambda b: (0, 0)),
    ],
    out_specs=pl.BlockSpec((1, DIM, TB), lambda b: (b, 0, 0)),
    out_shape=jax.ShapeDtypeStruct((B, DIM, TB), jnp.float32),
    compiler_params=pltpu.CompilerParams(dimension_semantics=("arbitrary",)),
)


@jax.jit
def _moe(tokens, router_w, router_b, f1w, f1b, f2w, f2b, scale):
    topi3, rank3, w0, w1, counts_f = _router(
        tokens, router_w, router_b.reshape(1, E))

    counts = counts_f[:, 0].astype(jnp.int32)
    nt = (counts + BLK - 1) // BLK
    tile_cum = jnp.cumsum(nt)
    start = (BLK * (tile_cum - nt)).astype(jnp.int32)
    t_idx = jnp.arange(T_TILES, dtype=jnp.int32)
    te_map = jnp.minimum(
        jnp.sum((t_idx[:, None] >= tile_cum[None, :]).astype(jnp.int32),
                axis=1), E - 1).astype(jnp.int32)
    te_plus = jnp.concatenate([te_map, tile_cum[-1:]])

    pos3 = _posmap(start, topi3, rank3)
    pos = pos3.reshape(NPAIR)
    xs = _dispatch(tokens, pos)
    yw = _gmm(te_plus, xs,
              f1w.astype(jnp.bfloat16), f1b.reshape(E, 1, HIDDEN),
              f2w.astype(jnp.bfloat16), f2b.reshape(E, 1, DIM))
    yg0, yg1 = _combine(yw, pos)
    return _finish(tokens, yg0, yg1, w0, w1, scale.reshape(1, 1))


def kernel(x, router_w, router_b, fc1_w, fc1_b, fc2_w, fc2_b, scale):
    b, c, h, w = x.shape
    tokens = jnp.transpose(x, (0, 2, 3, 1)).reshape(b * h * w, c)
    out = _moe(tokens, router_w, router_b, fc1_w, fc1_b, fc2_w, fc2_b,
               scale)
    return out.reshape(b, c, h, w)


# speedup vs baseline: 1.1532x; 1.0738x over previous
import functools

import jax
import jax.numpy as jnp
from jax import lax
from jax.experimental import pallas as pl
from jax.experimental.pallas import tpu as pltpu
from jax.experimental.pallas import tpu_sc as plsc

B, DIM, H, W = 8, 768, 32, 32
E, K = 8, 2
HIDDEN = DIM * 2
N_TOK = B * H * W
TB = 1024
NB = N_TOK // TB
NPAIR = K * N_TOK
BLK = 512
T_TILES = NPAIR // BLK + E
NROWS = T_TILES * BLK

NC, NS = 2, 16
NWK = NC * NS
CH = 64
CHC = 32

_INV_SQRT2 = 0.7071067811865476


def _router_body(tok_ref, rw_ref, rb_ref,
                 topi_ref, rank_ref, w0_ref, w1_ref, counts_ref,
                 carry_ref, u_ref, ident_ref):
    b = pl.program_id(0)

    @pl.when(b == 0)
    def _():
        carry_ref[...] = jnp.zeros_like(carry_ref)
        ii2 = lax.broadcasted_iota(jnp.int32, (2 * TB, 2 * TB), 0)
        jj2 = lax.broadcasted_iota(jnp.int32, (2 * TB, 2 * TB), 1)
        u_ref[...] = (ii2 < jj2).astype(jnp.bfloat16)
        ii1 = lax.broadcasted_iota(jnp.int32, (TB, TB), 0)
        jj1 = lax.broadcasted_iota(jnp.int32, (TB, TB), 1)
        ident_ref[...] = (ii1 == jj1).astype(jnp.float32)

    xb = tok_ref[...]
    logits = lax.dot_general(xb, rw_ref[...],
                             (((1,), (1,)), ((), ())),
                             preferred_element_type=jnp.float32)
    logits = logits + rb_ref[...]
    idx = lax.broadcasted_iota(jnp.int32, (TB, E), 1)
    m1 = jnp.max(logits, axis=1, keepdims=True)
    i1 = jnp.min(jnp.where(logits == m1, idx, E), axis=1, keepdims=True)
    l2 = jnp.where(idx == i1, -jnp.inf, logits)
    m2 = jnp.max(l2, axis=1, keepdims=True)
    i2 = jnp.min(jnp.where(l2 == m2, idx, E), axis=1, keepdims=True)
    e21 = jnp.exp(m2 - m1)
    w1 = 1.0 / (1.0 + e21)
    w2 = 1.0 - w1
    w0_ref[...] = w1
    w1_ref[...] = w2

    ident = ident_ref[...]
    i1r = lax.dot_general(i1.astype(jnp.float32), ident,
                          (((0,), (0,)), ((), ())),
                          preferred_element_type=jnp.float32)
    i2r = lax.dot_general(i2.astype(jnp.float32), ident,
                          (((0,), (0,)), ((), ())),
                          preferred_element_type=jnp.float32)
    ir = jnp.concatenate([i1r, i2r], axis=1).astype(jnp.int32)

    idx_e = lax.broadcasted_iota(jnp.int32, (E, 2 * TB), 0)
    mt = (idx_e == ir).astype(jnp.float32)
    prefix = lax.dot_general(mt.astype(jnp.bfloat16), u_ref[...],
                             (((1,), (0,)), ((), ())),
                             preferred_element_type=jnp.float32)
    prefix = prefix + carry_ref[...]
    rank_row = jnp.sum(mt * prefix, axis=0, keepdims=True)
    carry_ref[...] += jnp.sum(mt, axis=1, keepdims=True)
    counts_ref[...] = carry_ref[...]

    topi_ref[...] = ir.reshape(1, 1, 2 * TB)
    rank_ref[...] = rank_row.astype(jnp.int32).reshape(1, 1, 2 * TB)


_router = pl.pallas_call(
    _router_body,
    grid=(NB,),
    in_specs=[
        pl.BlockSpec((TB, DIM), lambda b: (b, 0)),
        pl.BlockSpec((E, DIM), lambda b: (0, 0)),
        pl.BlockSpec((1, E), lambda b: (0, 0)),
    ],
    out_specs=[
        pl.BlockSpec((1, 1, 2 * TB), lambda b: (b, 0, 0)),
        pl.BlockSpec((1, 1, 2 * TB), lambda b: (b, 0, 0)),
        pl.BlockSpec((TB, 1), lambda b: (b, 0)),
        pl.BlockSpec((TB, 1), lambda b: (b, 0)),
        pl.BlockSpec((E, 1), lambda b: (0, 0)),
    ],
    out_shape=[
        jax.ShapeDtypeStruct((NB, 1, 2 * TB), jnp.int32),
        jax.ShapeDtypeStruct((NB, 1, 2 * TB), jnp.int32),
        jax.ShapeDtypeStruct((N_TOK, 1), jnp.float32),
        jax.ShapeDtypeStruct((N_TOK, 1), jnp.float32),
        jax.ShapeDtypeStruct((E, 1), jnp.float32),
    ],
    scratch_shapes=[
        pltpu.VMEM((E, 1), jnp.float32),
        pltpu.VMEM((2 * TB, 2 * TB), jnp.bfloat16),
        pltpu.VMEM((TB, TB), jnp.float32),
    ],
    compiler_params=pltpu.CompilerParams(dimension_semantics=("arbitrary",)),
)


def _posmap_body(start_ref, topi_ref, rank_ref, pos_ref):
    t = topi_ref[...]
    acc = rank_ref[...]
    for e in range(E):
        acc = acc + jnp.where(t == e, start_ref[e], 0)
    pos_ref[...] = acc


_posmap = pl.pallas_call(
    _posmap_body,
    grid_spec=pltpu.PrefetchScalarGridSpec(
        num_scalar_prefetch=1,
        grid=(NB,),
        in_specs=[
            pl.BlockSpec((1, 1, 2 * TB), lambda b, s: (b, 0, 0)),
            pl.BlockSpec((1, 1, 2 * TB), lambda b, s: (b, 0, 0)),
        ],
        out_specs=pl.BlockSpec((1, 1, 2 * TB), lambda b, s: (b, 0, 0)),
    ),
    out_shape=jax.ShapeDtypeStruct((NB, 1, 2 * TB), jnp.int32),
    compiler_params=pltpu.CompilerParams(dimension_semantics=("arbitrary",)),
)


_sc_mesh = plsc.VectorSubcoreMesh(core_axis_name="c", subcore_axis_name="s")

_PW = NPAIR // NWK
_NCH = _PW // CH


@functools.partial(
    pl.kernel,
    mesh=_sc_mesh,
    out_type=jax.ShapeDtypeStruct((NROWS, DIM), jnp.float32),
    scratch_types=[
        pltpu.VMEM((_PW,), jnp.int32),
        pltpu.VMEM((CH,), jnp.int32),
        pltpu.VMEM((CH,), jnp.int32),
        pltpu.VMEM((CH,), jnp.int32),
        pltpu.VMEM((CH,), jnp.int32),
        pltpu.VMEM((CH, DIM), jnp.float32),
        pltpu.VMEM((CH, DIM), jnp.float32),
        pltpu.SemaphoreType.DMA,
        pltpu.SemaphoreType.DMA,
        pltpu.SemaphoreType.DMA,
        pltpu.SemaphoreType.DMA,
    ],
)
def _dispatch(tok_hbm, pos_hbm, xs_hbm,
              pos_all, pos_c0, pos_c1, tok_c0, tok_c1, row0, row1,
              g0, g1, s0, s1):
    wid = lax.axis_index("s") * NC + lax.axis_index("c")
    base = wid * _PW
    pos_c = [pos_c0, pos_c1]
    tok_c = [tok_c0, tok_c1]
    row = [row0, row1]
    gsem = [g0, g1]
    ssem = [s0, s1]
    pltpu.sync_copy(pos_hbm.at[pl.ds(base, _PW)], pos_all)

    def fill(bi, c):
        for j in range(CH // 16):
            sl = pl.ds(16 * j, 16)
            pos_c[bi][sl] = pos_all[pl.ds(c * CH + 16 * j, 16)]
            pvec = jnp.full((16,), base + c * CH + 16 * j, jnp.int32) + \
                lax.iota(jnp.int32, 16)
            tok_c[bi][sl] = ((pvec >> 11) << 10) + (pvec & (TB - 1))

    d_g = [None, None]
    d_s = [None, None]
    fill(0, 0)
    d_g[0] = pltpu.async_copy(tok_hbm.at[tok_c[0]], row[0], gsem[0])
    for c in range(_NCH):
        cb = c % 2
        nb = (c + 1) % 2
        if c + 1 < _NCH:
            if d_s[nb] is not None:
                d_s[nb].wait()
            fill(nb, c + 1)
            d_g[nb] = pltpu.async_copy(tok_hbm.at[tok_c[nb]], row[nb],
                                       gsem[nb])
        d_g[cb].wait()
        d_s[cb] = pltpu.async_copy(row[cb], xs_hbm.at[pos_c[cb]], ssem[cb])
    d_s[0].wait()
    d_s[1].wait()


def _gmm_body(te_ref, xs_ref, f1w_ref, f1b_ref, f2w_ref, f2b_ref, out_ref):
    @pl.when(pl.program_id(0) < te_ref[T_TILES])
    def _():
        xb = xs_ref[...].astype(jnp.bfloat16)
        h = lax.dot_general(xb, f1w_ref[0],
                            (((1,), (1,)), ((), ())),
                            preferred_element_type=jnp.float32)
        h = h + f1b_ref[0]
        h = 0.5 * h * (1.0 + lax.erf(h * _INV_SQRT2))
        y = lax.dot_general(h.astype(jnp.bfloat16), f2w_ref[0],
                            (((1,), (1,)), ((), ())),
                            preferred_element_type=jnp.float32)
        out_ref[...] = y + f2b_ref[0]


_gmm = pl.pallas_call(
    _gmm_body,
    grid_spec=pltpu.PrefetchScalarGridSpec(
        num_scalar_prefetch=1,
        grid=(T_TILES,),
        in_specs=[
            pl.BlockSpec((BLK, DIM), lambda t, te: (t, 0)),
            pl.BlockSpec((1, HIDDEN, DIM), lambda t, te: (te[t], 0, 0)),
            pl.BlockSpec((1, 1, HIDDEN), lambda t, te: (te[t], 0, 0)),
            pl.BlockSpec((1, DIM, HIDDEN), lambda t, te: (te[t], 0, 0)),
            pl.BlockSpec((1, 1, DIM), lambda t, te: (te[t], 0, 0)),
        ],
        out_specs=pl.BlockSpec((BLK, DIM), lambda t, te: (t, 0)),
    ),
    out_shape=jax.ShapeDtypeStruct((NROWS, DIM), jnp.float32),
    compiler_params=pltpu.CompilerParams(dimension_semantics=("arbitrary",)),
)


_TW = N_TOK // NWK
_NCHC = _TW // CHC


@functools.partial(
    pl.kernel,
    mesh=_sc_mesh,
    out_type=(
        jax.ShapeDtypeStruct((N_TOK, DIM), jnp.float32),
        jax.ShapeDtypeStruct((N_TOK, DIM), jnp.float32),
    ),
    scratch_types=[
        pltpu.VMEM((_TW,), jnp.int32),
        pltpu.VMEM((_TW,), jnp.int32),
        pltpu.VMEM((CHC,), jnp.int32),
        pltpu.VMEM((CHC,), jnp.int32),
        pltpu.VMEM((CHC,), jnp.int32),
        pltpu.VMEM((CHC,), jnp.int32),
        pltpu.VMEM((CHC, DIM), jnp.float32),
        pltpu.VMEM((CHC, DIM), jnp.float32),
        pltpu.VMEM((CHC, DIM), jnp.float32),
        pltpu.VMEM((CHC, DIM), jnp.float32),
        pltpu.SemaphoreType.DMA,
        pltpu.SemaphoreType.DMA,
        pltpu.SemaphoreType.DMA,
        pltpu.SemaphoreType.DMA,
        pltpu.SemaphoreType.DMA,
        pltpu.SemaphoreType.DMA,
        pltpu.SemaphoreType.DMA,
        pltpu.SemaphoreType.DMA,
    ],
)
def _combine(yw_hbm, pos_hbm, yg0_hbm, yg1_hbm,
             p0_all, p1_all, i00, i01, i10, i11,
             b00, b01, b10, b11,
             g00, g01, g10, g11, s00, s01, s10, s11):
    wid = lax.axis_index("s") * NC + lax.axis_index("c")
    tbase = wid * _TW
    p0base = (tbase // TB) * (2 * TB) + (tbase % TB)
    p1base = p0base + TB
    pltpu.sync_copy(pos_hbm.at[pl.ds(p0base, _TW)], p0_all)
    pltpu.sync_copy(pos_hbm.at[pl.ds(p1base, _TW)], p1_all)

    p_all = [p0_all, p1_all]
    idx = [[i00, i01], [i10, i11]]
    buf = [[b00, b01], [b10, b11]]
    gsem = [[g00, g01], [g10, g11]]
    ssem = [[s00, s01], [s10, s11]]
    out = [yg0_hbm, yg1_hbm]

    def fill(k, bi, c):
        for j in range(CHC // 16):
            sl = pl.ds(16 * j, 16)
            idx[k][bi][sl] = p_all[k][pl.ds(c * CHC + 16 * j, 16)]

    d_g = [[None, None], [None, None]]
    d_s = [[None, None], [None, None]]
    for k in (0, 1):
        fill(k, 0, 0)
        d_g[k][0] = pltpu.async_copy(yw_hbm.at[idx[k][0]], buf[k][0],
                                     gsem[k][0])
    for c in range(_NCHC):
        cb = c % 2
        nb = (c + 1) % 2
        for k in (0, 1):
            if c + 1 < _NCHC:
                if d_s[k][nb] is not None:
                    d_s[k][nb].wait()
                fill(k, nb, c + 1)
                d_g[k][nb] = pltpu.async_copy(yw_hbm.at[idx[k][nb]],
                                              buf[k][nb], gsem[k][nb])
            d_g[k][cb].wait()
            d_s[k][cb] = pltpu.async_copy(
                buf[k][cb],
                out[k].at[pl.ds(tbase + c * CHC, CHC), :], ssem[k][cb])
    for k in (0, 1):
        d_s[k][0].wait()
        d_s[k][1].wait()


def _finish_body(x_ref, a_ref, b_ref, w0_ref, w1_ref, scale_ref, out_ref):
    s = scale_ref[0, 0]
    out_ref[...] = x_ref[...] + s * (w0_ref[...] * a_ref[...] +
                                     w1_ref[...] * b_ref[...])


_finish = pl.pallas_call(
    _finish_body,
    grid=(NB,),
    in_specs=[
        pl.BlockSpec((TB, DIM), lambda b: (b, 0)),
        pl.BlockSpec((TB, DIM), lambda b: (b, 0)),
        pl.BlockSpec((TB, DIM), lambda b: (b, 0)),
        pl.BlockSpec((TB, 1), lambda b: (b, 0)),
        pl.BlockSpec((TB, 1), lambda b: (b, 0)),
        pl.BlockSpec((1, 1), lambda b: (0, 0)),
    ],
    out_specs=pl.BlockSpec((TB, DIM), lambda b: (b, 0)),
    out_shape=jax.ShapeDtypeStruct((N_TOK, DIM), jnp.float32),
    compiler_params=pltpu.CompilerParams(dimension_semantics=("arbitrary",)),
)


def _wcast_body(f1_ref, f2_ref, o1_ref, o2_ref):
    o1_ref[...] = f1_ref[...].astype(jnp.bfloat16)
    o2_ref[...] = f2_ref[...].astype(jnp.bfloat16)


_wcast = pl.pallas_call(
    _wcast_body,
    grid=(E,),
    in_specs=[
        pl.BlockSpec((1, HIDDEN, DIM), lambda e: (e, 0, 0)),
        pl.BlockSpec((1, DIM, HIDDEN), lambda e: (e, 0, 0)),
    ],
    out_specs=[
        pl.BlockSpec((1, HIDDEN, DIM), lambda e: (e, 0, 0)),
        pl.BlockSpec((1, DIM, HIDDEN), lambda e: (e, 0, 0)),
    ],
    out_shape=[
        jax.ShapeDtypeStruct((E, HIDDEN, DIM), jnp.bfloat16),
        jax.ShapeDtypeStruct((E, DIM, HIDDEN), jnp.bfloat16),
    ],
    compiler_params=pltpu.CompilerParams(dimension_semantics=("arbitrary",)),
)


@jax.jit
def _moe(tokens, router_w, router_b, f1w, f1b, f2w, f2b, scale):
    topi3, rank3, w0, w1, counts_f = _router(
        tokens, router_w, router_b.reshape(1, E))

    counts = counts_f[:, 0].astype(jnp.int32)
    nt = (counts + BLK - 1) // BLK
    tile_cum = jnp.cumsum(nt)
    start = (BLK * (tile_cum - nt)).astype(jnp.int32)
    t_idx = jnp.arange(T_TILES, dtype=jnp.int32)
    te_map = jnp.minimum(
        jnp.sum((t_idx[:, None] >= tile_cum[None, :]).astype(jnp.int32),
                axis=1), E - 1).astype(jnp.int32)
    te_plus = jnp.concatenate([te_map, tile_cum[-1:]])

    pos3 = _posmap(start, topi3, rank3)
    pos = pos3.reshape(NPAIR)
    f1w_bf, f2w_bf = _wcast(f1w, f2w)
    xs = _dispatch(tokens, pos)
    yw = _gmm(te_plus, xs,
              f1w_bf, f1b.reshape(E, 1, HIDDEN),
              f2w_bf, f2b.reshape(E, 1, DIM))
    yg0, yg1 = _combine(yw, pos)
    return _finish(tokens, yg0, yg1, w0, w1, scale.reshape(1, 1))


def kernel(x, router_w, router_b, fc1_w, fc1_b, fc2_w, fc2_b, scale):
    b, c, h, w = x.shape
    tokens = jnp.transpose(x, (0, 2, 3, 1)).reshape(b * h * w, c)
    out = _moe(tokens, router_w, router_b, fc1_w, fc1_b, fc2_w, fc2_b,
               scale)
    return jnp.transpose(out.reshape(b, h, w, c), (0, 3, 1, 2))


# speedup vs baseline: 1.3410x; 1.1628x over previous
import functools

import jax
import jax.numpy as jnp
from jax import lax
from jax.experimental import pallas as pl
from jax.experimental.pallas import tpu as pltpu
from jax.experimental.pallas import tpu_sc as plsc

B, DIM, H, W = 8, 768, 32, 32
E, K = 8, 2
HIDDEN = DIM * 2
N_TOK = B * H * W
TB = 1024
NB = N_TOK // TB
NPAIR = K * N_TOK
BLK = 512
T_TILES = NPAIR // BLK + E
NROWS = T_TILES * BLK

NC, NS = 2, 16
NWK = NC * NS
CH = 64
CHC = 32

_INV_SQRT2 = 0.7071067811865476


def _router_body(tok_ref, rw_ref, rb_ref,
                 topi_ref, rank_ref, w0_ref, w1_ref, counts_ref,
                 carry_ref, u_ref, ident_ref):
    b = pl.program_id(0)

    @pl.when(b == 0)
    def _():
        carry_ref[...] = jnp.zeros_like(carry_ref)
        ii2 = lax.broadcasted_iota(jnp.int32, (2 * TB, 2 * TB), 0)
        jj2 = lax.broadcasted_iota(jnp.int32, (2 * TB, 2 * TB), 1)
        u_ref[...] = (ii2 < jj2).astype(jnp.bfloat16)
        ii1 = lax.broadcasted_iota(jnp.int32, (TB, TB), 0)
        jj1 = lax.broadcasted_iota(jnp.int32, (TB, TB), 1)
        ident_ref[...] = (ii1 == jj1).astype(jnp.float32)

    xb = tok_ref[...]
    logits = lax.dot_general(xb, rw_ref[...],
                             (((1,), (1,)), ((), ())),
                             preferred_element_type=jnp.float32)
    logits = logits + rb_ref[...]
    idx = lax.broadcasted_iota(jnp.int32, (TB, E), 1)
    m1 = jnp.max(logits, axis=1, keepdims=True)
    i1 = jnp.min(jnp.where(logits == m1, idx, E), axis=1, keepdims=True)
    l2 = jnp.where(idx == i1, -jnp.inf, logits)
    m2 = jnp.max(l2, axis=1, keepdims=True)
    i2 = jnp.min(jnp.where(l2 == m2, idx, E), axis=1, keepdims=True)
    e21 = jnp.exp(m2 - m1)
    w1 = 1.0 / (1.0 + e21)
    w2 = 1.0 - w1
    w0_ref[...] = w1
    w1_ref[...] = w2

    ident = ident_ref[...]
    i1r = lax.dot_general(i1.astype(jnp.float32), ident,
                          (((0,), (0,)), ((), ())),
                          preferred_element_type=jnp.float32)
    i2r = lax.dot_general(i2.astype(jnp.float32), ident,
                          (((0,), (0,)), ((), ())),
                          preferred_element_type=jnp.float32)
    ir = jnp.concatenate([i1r, i2r], axis=1).astype(jnp.int32)

    idx_e = lax.broadcasted_iota(jnp.int32, (E, 2 * TB), 0)
    mt = (idx_e == ir).astype(jnp.float32)
    prefix = lax.dot_general(mt.astype(jnp.bfloat16), u_ref[...],
                             (((1,), (0,)), ((), ())),
                             preferred_element_type=jnp.float32)
    prefix = prefix + carry_ref[...]
    rank_row = jnp.sum(mt * prefix, axis=0, keepdims=True)
    carry_ref[...] += jnp.sum(mt, axis=1, keepdims=True)
    counts_ref[...] = carry_ref[...]

    topi_ref[...] = ir.reshape(1, 1, 2 * TB)
    rank_ref[...] = rank_row.astype(jnp.int32).reshape(1, 1, 2 * TB)


_router = pl.pallas_call(
    _router_body,
    grid=(NB,),
    in_specs=[
        pl.BlockSpec((TB, DIM), lambda b: (b, 0)),
        pl.BlockSpec((E, DIM), lambda b: (0, 0)),
        pl.BlockSpec((1, E), lambda b: (0, 0)),
    ],
    out_specs=[
        pl.BlockSpec((1, 1, 2 * TB), lambda b: (b, 0, 0)),
        pl.BlockSpec((1, 1, 2 * TB), lambda b: (b, 0, 0)),
        pl.BlockSpec((TB, 1), lambda b: (b, 0)),
        pl.BlockSpec((TB, 1), lambda b: (b, 0)),
        pl.BlockSpec((E, 1), lambda b: (0, 0)),
    ],
    out_shape=[
        jax.ShapeDtypeStruct((NB, 1, 2 * TB), jnp.int32),
        jax.ShapeDtypeStruct((NB, 1, 2 * TB), jnp.int32),
        jax.ShapeDtypeStruct((N_TOK, 1), jnp.float32),
        jax.ShapeDtypeStruct((N_TOK, 1), jnp.float32),
        jax.ShapeDtypeStruct((E, 1), jnp.float32),
    ],
    scratch_shapes=[
        pltpu.VMEM((E, 1), jnp.float32),
        pltpu.VMEM((2 * TB, 2 * TB), jnp.bfloat16),
        pltpu.VMEM((TB, TB), jnp.float32),
    ],
    compiler_params=pltpu.CompilerParams(dimension_semantics=("arbitrary",)),
)


def _posmap_body(start_ref, topi_ref, rank_ref, pos_ref):
    t = topi_ref[...]
    acc = rank_ref[...]
    for e in range(E):
        acc = acc + jnp.where(t == e, start_ref[e], 0)
    pos_ref[...] = acc


_posmap = pl.pallas_call(
    _posmap_body,
    grid_spec=pltpu.PrefetchScalarGridSpec(
        num_scalar_prefetch=1,
        grid=(NB,),
        in_specs=[
            pl.BlockSpec((1, 1, 2 * TB), lambda b, s: (b, 0, 0)),
            pl.BlockSpec((1, 1, 2 * TB), lambda b, s: (b, 0, 0)),
        ],
        out_specs=pl.BlockSpec((1, 1, 2 * TB), lambda b, s: (b, 0, 0)),
    ),
    out_shape=jax.ShapeDtypeStruct((NB, 1, 2 * TB), jnp.int32),
    compiler_params=pltpu.CompilerParams(dimension_semantics=("arbitrary",)),
)


_sc_mesh = plsc.VectorSubcoreMesh(core_axis_name="c", subcore_axis_name="s")

_PW = NPAIR // NWK
_NCH = _PW // CH


@functools.partial(
    pl.kernel,
    mesh=_sc_mesh,
    out_type=jax.ShapeDtypeStruct((NROWS, DIM), jnp.float32),
    scratch_types=[
        pltpu.VMEM((_PW,), jnp.int32),
        pltpu.VMEM((CH,), jnp.int32),
        pltpu.VMEM((CH,), jnp.int32),
        pltpu.VMEM((CH,), jnp.int32),
        pltpu.VMEM((CH,), jnp.int32),
        pltpu.VMEM((CH, DIM), jnp.float32),
        pltpu.VMEM((CH, DIM), jnp.float32),
        pltpu.SemaphoreType.DMA,
        pltpu.SemaphoreType.DMA,
        pltpu.SemaphoreType.DMA,
        pltpu.SemaphoreType.DMA,
    ],
)
def _dispatch(tok_hbm, pos_hbm, xs_hbm,
              pos_all, pos_c0, pos_c1, tok_c0, tok_c1, row0, row1,
              g0, g1, s0, s1):
    wid = lax.axis_index("s") * NC + lax.axis_index("c")
    base = wid * _PW
    pos_c = [pos_c0, pos_c1]
    tok_c = [tok_c0, tok_c1]
    row = [row0, row1]
    gsem = [g0, g1]
    ssem = [s0, s1]
    pltpu.sync_copy(pos_hbm.at[pl.ds(base, _PW)], pos_all)

    def fill(bi, c):
        for j in range(CH // 16):
            sl = pl.ds(16 * j, 16)
            pos_c[bi][sl] = pos_all[pl.ds(c * CH + 16 * j, 16)]
            pvec = jnp.full((16,), base + c * CH + 16 * j, jnp.int32) + \
                lax.iota(jnp.int32, 16)
            tok_c[bi][sl] = ((pvec >> 11) << 10) + (pvec & (TB - 1))

    d_g = [None, None]
    d_s = [None, None]
    fill(0, 0)
    d_g[0] = pltpu.async_copy(tok_hbm.at[tok_c[0]], row[0], gsem[0])
    for c in range(_NCH):
        cb = c % 2
        nb = (c + 1) % 2
        if c + 1 < _NCH:
            if d_s[nb] is not None:
                d_s[nb].wait()
            fill(nb, c + 1)
            d_g[nb] = pltpu.async_copy(tok_hbm.at[tok_c[nb]], row[nb],
                                       gsem[nb])
        d_g[cb].wait()
        d_s[cb] = pltpu.async_copy(row[cb], xs_hbm.at[pos_c[cb]], ssem[cb])
    d_s[0].wait()
    d_s[1].wait()


def _gmm_body(te_ref, xs_ref, f1w_ref, f1b_ref, f2w_ref, f2b_ref, out_ref,
              w1c_ref, w2c_ref, prev_ref):
    t = pl.program_id(0)
    e = te_ref[t]

    @pl.when(jnp.logical_or(t == 0, e != prev_ref[0]))
    def _():
        w1c_ref[...] = f1w_ref[0].astype(jnp.bfloat16)
        w2c_ref[...] = f2w_ref[0].astype(jnp.bfloat16)

    prev_ref[0] = e

    @pl.when(t < te_ref[T_TILES])
    def _():
        xb = xs_ref[...].astype(jnp.bfloat16)
        h = lax.dot_general(xb, w1c_ref[...],
                            (((1,), (1,)), ((), ())),
                            preferred_element_type=jnp.float32)
        h = h + f1b_ref[0]
        h = 0.5 * h * (1.0 + lax.erf(h * _INV_SQRT2))
        y = lax.dot_general(h.astype(jnp.bfloat16), w2c_ref[...],
                            (((1,), (1,)), ((), ())),
                            preferred_element_type=jnp.float32)
        y = y + f2b_ref[0]

        def bfbits(v):
            vi = lax.bitcast_convert_type(v, jnp.int32)
            r = (vi + 0x7FFF + ((vi >> 16) & 1)) >> 16
            return r & 0xFFFF

        lo = bfbits(y[:, :DIM // 2])
        hi = bfbits(y[:, DIM // 2:])
        out_ref[...] = lo | (hi << 16)


_gmm = pl.pallas_call(
    _gmm_body,
    grid_spec=pltpu.PrefetchScalarGridSpec(
        num_scalar_prefetch=1,
        grid=(T_TILES,),
        in_specs=[
            pl.BlockSpec((BLK, DIM), lambda t, te: (t, 0)),
            pl.BlockSpec((1, HIDDEN, DIM), lambda t, te: (te[t], 0, 0)),
            pl.BlockSpec((1, 1, HIDDEN), lambda t, te: (te[t], 0, 0)),
            pl.BlockSpec((1, DIM, HIDDEN), lambda t, te: (te[t], 0, 0)),
            pl.BlockSpec((1, 1, DIM), lambda t, te: (te[t], 0, 0)),
        ],
        out_specs=pl.BlockSpec((BLK, DIM // 2), lambda t, te: (t, 0)),
        scratch_shapes=[
            pltpu.VMEM((HIDDEN, DIM), jnp.bfloat16),
            pltpu.VMEM((DIM, HIDDEN), jnp.bfloat16),
            pltpu.SMEM((1,), jnp.int32),
        ],
    ),
    out_shape=jax.ShapeDtypeStruct((NROWS, DIM // 2), jnp.int32),
    compiler_params=pltpu.CompilerParams(dimension_semantics=("arbitrary",)),
)


_TW = N_TOK // NWK
_NCHC = _TW // CHC


@functools.partial(
    pl.kernel,
    mesh=_sc_mesh,
    out_type=(
        jax.ShapeDtypeStruct((N_TOK, DIM // 2), jnp.int32),
        jax.ShapeDtypeStruct((N_TOK, DIM // 2), jnp.int32),
    ),
    scratch_types=[
        pltpu.VMEM((_TW,), jnp.int32),
        pltpu.VMEM((_TW,), jnp.int32),
        pltpu.VMEM((CHC,), jnp.int32),
        pltpu.VMEM((CHC,), jnp.int32),
        pltpu.VMEM((CHC,), jnp.int32),
        pltpu.VMEM((CHC,), jnp.int32),
        pltpu.VMEM((CHC, DIM // 2), jnp.int32),
        pltpu.VMEM((CHC, DIM // 2), jnp.int32),
        pltpu.VMEM((CHC, DIM // 2), jnp.int32),
        pltpu.VMEM((CHC, DIM // 2), jnp.int32),
        pltpu.SemaphoreType.DMA,
        pltpu.SemaphoreType.DMA,
        pltpu.SemaphoreType.DMA,
        pltpu.SemaphoreType.DMA,
        pltpu.SemaphoreType.DMA,
        pltpu.SemaphoreType.DMA,
        pltpu.SemaphoreType.DMA,
        pltpu.SemaphoreType.DMA,
    ],
)
def _combine(yw_hbm, pos_hbm, yg0_hbm, yg1_hbm,
             p0_all, p1_all, i00, i01, i10, i11,
             b00, b01, b10, b11,
             g00, g01, g10, g11, s00, s01, s10, s11):
    wid = lax.axis_index("s") * NC + lax.axis_index("c")
    tbase = wid * _TW
    p0base = (tbase // TB) * (2 * TB) + (tbase % TB)
    p1base = p0base + TB
    pltpu.sync_copy(pos_hbm.at[pl.ds(p0base, _TW)], p0_all)
    pltpu.sync_copy(pos_hbm.at[pl.ds(p1base, _TW)], p1_all)

    p_all = [p0_all, p1_all]
    idx = [[i00, i01], [i10, i11]]
    buf = [[b00, b01], [b10, b11]]
    gsem = [[g00, g01], [g10, g11]]
    ssem = [[s00, s01], [s10, s11]]
    out = [yg0_hbm, yg1_hbm]

    def fill(k, bi, c):
        for j in range(CHC // 16):
            sl = pl.ds(16 * j, 16)
            idx[k][bi][sl] = p_all[k][pl.ds(c * CHC + 16 * j, 16)]

    d_g = [[None, None], [None, None]]
    d_s = [[None, None], [None, None]]
    for k in (0, 1):
        fill(k, 0, 0)
        d_g[k][0] = pltpu.async_copy(yw_hbm.at[idx[k][0]], buf[k][0],
                                     gsem[k][0])
    for c in range(_NCHC):
        cb = c % 2
        nb = (c + 1) % 2
        for k in (0, 1):
            if c + 1 < _NCHC:
                if d_s[k][nb] is not None:
                    d_s[k][nb].wait()
                fill(k, nb, c + 1)
                d_g[k][nb] = pltpu.async_copy(yw_hbm.at[idx[k][nb]],
                                              buf[k][nb], gsem[k][nb])
            d_g[k][cb].wait()
            d_s[k][cb] = pltpu.async_copy(
                buf[k][cb],
                out[k].at[pl.ds(tbase + c * CHC, CHC), :], ssem[k][cb])
    for k in (0, 1):
        d_s[k][0].wait()
        d_s[k][1].wait()


def _unpack(p):
    lo = lax.bitcast_convert_type(p << 16, jnp.float32)
    hi = lax.bitcast_convert_type(p & jnp.int32(-65536), jnp.float32)
    return jnp.concatenate([lo, hi], axis=1)


def _finish_body(x_ref, a_ref, b_ref, w0_ref, w1_ref, scale_ref, out_ref):
    s = scale_ref[0, 0]
    a = _unpack(a_ref[...])
    bb = _unpack(b_ref[...])
    out_ref[...] = x_ref[...] + s * (w0_ref[...] * a + w1_ref[...] * bb)


_finish = pl.pallas_call(
    _finish_body,
    grid=(NB,),
    in_specs=[
        pl.BlockSpec((TB, DIM), lambda b: (b, 0)),
        pl.BlockSpec((TB, DIM // 2), lambda b: (b, 0)),
        pl.BlockSpec((TB, DIM // 2), lambda b: (b, 0)),
        pl.BlockSpec((TB, 1), lambda b: (b, 0)),
        pl.BlockSpec((TB, 1), lambda b: (b, 0)),
        pl.BlockSpec((1, 1), lambda b: (0, 0)),
    ],
    out_specs=pl.BlockSpec((TB, DIM), lambda b: (b, 0)),
    out_shape=jax.ShapeDtypeStruct((N_TOK, DIM), jnp.float32),
    compiler_params=pltpu.CompilerParams(dimension_semantics=("arbitrary",)),
)


@jax.jit
def _moe(tokens, router_w, router_b, f1w, f1b, f2w, f2b, scale):
    topi3, rank3, w0, w1, counts_f = _router(
        tokens, router_w, router_b.reshape(1, E))

    counts = counts_f[:, 0].astype(jnp.int32)
    nt = (counts + BLK - 1) // BLK
    tile_cum = jnp.cumsum(nt)
    start = (BLK * (tile_cum - nt)).astype(jnp.int32)
    t_idx = jnp.arange(T_TILES, dtype=jnp.int32)
    te_map = jnp.minimum(
        jnp.sum((t_idx[:, None] >= tile_cum[None, :]).astype(jnp.int32),
                axis=1), E - 1).astype(jnp.int32)
    te_plus = jnp.concatenate([te_map, tile_cum[-1:]])

    pos3 = _posmap(start, topi3, rank3)
    pos = pos3.reshape(NPAIR)
    xs = _dispatch(tokens, pos)
    yw = _gmm(te_plus, xs,
              f1w, f1b.reshape(E, 1, HIDDEN),
              f2w, f2b.reshape(E, 1, DIM))
    yg0, yg1 = _combine(yw, pos)
    return _finish(tokens, yg0, yg1, w0, w1, scale.reshape(1, 1))


def kernel(x, router_w, router_b, fc1_w, fc1_b, fc2_w, fc2_b, scale):
    b, c, h, w = x.shape
    tokens = jnp.transpose(x, (0, 2, 3, 1)).reshape(b * h * w, c)
    out = _moe(tokens, router_w, router_b, fc1_w, fc1_b, fc2_w, fc2_b,
               scale)
    return jnp.transpose(out.reshape(b, h, w, c), (0, 3, 1, 2))


# speedup vs baseline: 1.4418x; 1.0752x over previous
import functools

import jax
import jax.numpy as jnp
from jax import lax
from jax.experimental import pallas as pl
from jax.experimental.pallas import tpu as pltpu
from jax.experimental.pallas import tpu_sc as plsc

B, DIM, H, W = 8, 768, 32, 32
E, K = 8, 2
HIDDEN = DIM * 2
N_TOK = B * H * W
TB = 1024
NB = N_TOK // TB
NPAIR = K * N_TOK
BLK = 512
T_TILES = NPAIR // BLK + E
NROWS = T_TILES * BLK

NC, NS = 2, 16
NWK = NC * NS
CH = 64
CHC = 32

_INV_SQRT2 = 0.7071067811865476


def _bfpack(y):
    def bfbits(v):
        vi = lax.bitcast_convert_type(v, jnp.int32)
        r = (vi + 0x7FFF + ((vi >> 16) & 1)) >> 16
        return r & 0xFFFF

    half = y.shape[1] // 2
    return bfbits(y[:, :half]) | (bfbits(y[:, half:]) << 16)


def _bfunpack_bf16(p):
    lo = lax.bitcast_convert_type(p << 16, jnp.float32)
    hi = lax.bitcast_convert_type(p & jnp.int32(-65536), jnp.float32)
    return jnp.concatenate([lo, hi], axis=1).astype(jnp.bfloat16)


def _router_body(tok_ref, rw_ref, rb_ref,
                 topi_ref, rank_ref, w0_ref, w1_ref, counts_ref, tokp_ref,
                 carry_ref, u_ref, ident_ref):
    b = pl.program_id(0)

    @pl.when(b == 0)
    def _():
        carry_ref[...] = jnp.zeros_like(carry_ref)
        ii2 = lax.broadcasted_iota(jnp.int32, (2 * TB, 2 * TB), 0)
        jj2 = lax.broadcasted_iota(jnp.int32, (2 * TB, 2 * TB), 1)
        u_ref[...] = (ii2 < jj2).astype(jnp.bfloat16)
        ii1 = lax.broadcasted_iota(jnp.int32, (TB, TB), 0)
        jj1 = lax.broadcasted_iota(jnp.int32, (TB, TB), 1)
        ident_ref[...] = (ii1 == jj1).astype(jnp.float32)

    xb = tok_ref[...]
    tokp_ref[...] = _bfpack(xb)
    logits = lax.dot_general(xb, rw_ref[...],
                             (((1,), (1,)), ((), ())),
                             preferred_element_type=jnp.float32)
    logits = logits + rb_ref[...]
    idx = lax.broadcasted_iota(jnp.int32, (TB, E), 1)
    m1 = jnp.max(logits, axis=1, keepdims=True)
    i1 = jnp.min(jnp.where(logits == m1, idx, E), axis=1, keepdims=True)
    l2 = jnp.where(idx == i1, -jnp.inf, logits)
    m2 = jnp.max(l2, axis=1, keepdims=True)
    i2 = jnp.min(jnp.where(l2 == m2, idx, E), axis=1, keepdims=True)
    e21 = jnp.exp(m2 - m1)
    w1 = 1.0 / (1.0 + e21)
    w2 = 1.0 - w1
    w0_ref[...] = w1
    w1_ref[...] = w2

    ident = ident_ref[...]
    i1r = lax.dot_general(i1.astype(jnp.float32), ident,
                          (((0,), (0,)), ((), ())),
                          preferred_element_type=jnp.float32)
    i2r = lax.dot_general(i2.astype(jnp.float32), ident,
                          (((0,), (0,)), ((), ())),
                          preferred_element_type=jnp.float32)
    ir = jnp.concatenate([i1r, i2r], axis=1).astype(jnp.int32)

    idx_e = lax.broadcasted_iota(jnp.int32, (E, 2 * TB), 0)
    mt = (idx_e == ir).astype(jnp.float32)
    prefix = lax.dot_general(mt.astype(jnp.bfloat16), u_ref[...],
                             (((1,), (0,)), ((), ())),
                             preferred_element_type=jnp.float32)
    prefix = prefix + carry_ref[...]
    rank_row = jnp.sum(mt * prefix, axis=0, keepdims=True)
    carry_ref[...] += jnp.sum(mt, axis=1, keepdims=True)
    counts_ref[...] = carry_ref[...]

    topi_ref[...] = ir.reshape(1, 1, 2 * TB)
    rank_ref[...] = rank_row.astype(jnp.int32).reshape(1, 1, 2 * TB)


_router = pl.pallas_call(
    _router_body,
    grid=(NB,),
    in_specs=[
        pl.BlockSpec((TB, DIM), lambda b: (b, 0)),
        pl.BlockSpec((E, DIM), lambda b: (0, 0)),
        pl.BlockSpec((1, E), lambda b: (0, 0)),
    ],
    out_specs=[
        pl.BlockSpec((1, 1, 2 * TB), lambda b: (b, 0, 0)),
        pl.BlockSpec((1, 1, 2 * TB), lambda b: (b, 0, 0)),
        pl.BlockSpec((TB, 1), lambda b: (b, 0)),
        pl.BlockSpec((TB, 1), lambda b: (b, 0)),
        pl.BlockSpec((E, 1), lambda b: (0, 0)),
        pl.BlockSpec((TB, DIM // 2), lambda b: (b, 0)),
    ],
    out_shape=[
        jax.ShapeDtypeStruct((NB, 1, 2 * TB), jnp.int32),
        jax.ShapeDtypeStruct((NB, 1, 2 * TB), jnp.int32),
        jax.ShapeDtypeStruct((N_TOK, 1), jnp.float32),
        jax.ShapeDtypeStruct((N_TOK, 1), jnp.float32),
        jax.ShapeDtypeStruct((E, 1), jnp.float32),
        jax.ShapeDtypeStruct((N_TOK, DIM // 2), jnp.int32),
    ],
    scratch_shapes=[
        pltpu.VMEM((E, 1), jnp.float32),
        pltpu.VMEM((2 * TB, 2 * TB), jnp.bfloat16),
        pltpu.VMEM((TB, TB), jnp.float32),
    ],
    compiler_params=pltpu.CompilerParams(dimension_semantics=("arbitrary",)),
)


def _posmap_body(start_ref, topi_ref, rank_ref, pos_ref):
    t = topi_ref[...]
    acc = rank_ref[...]
    for e in range(E):
        acc = acc + jnp.where(t == e, start_ref[e], 0)
    pos_ref[...] = acc


_posmap = pl.pallas_call(
    _posmap_body,
    grid_spec=pltpu.PrefetchScalarGridSpec(
        num_scalar_prefetch=1,
        grid=(NB,),
        in_specs=[
            pl.BlockSpec((1, 1, 2 * TB), lambda b, s: (b, 0, 0)),
            pl.BlockSpec((1, 1, 2 * TB), lambda b, s: (b, 0, 0)),
        ],
        out_specs=pl.BlockSpec((1, 1, 2 * TB), lambda b, s: (b, 0, 0)),
    ),
    out_shape=jax.ShapeDtypeStruct((NB, 1, 2 * TB), jnp.int32),
    compiler_params=pltpu.CompilerParams(dimension_semantics=("arbitrary",)),
)


_sc_mesh = plsc.VectorSubcoreMesh(core_axis_name="c", subcore_axis_name="s")

_PW = NPAIR // NWK
_NCH = _PW // CH


@functools.partial(
    pl.kernel,
    mesh=_sc_mesh,
    out_type=jax.ShapeDtypeStruct((NROWS, DIM // 2), jnp.int32),
    scratch_types=[
        pltpu.VMEM((_PW,), jnp.int32),
        pltpu.VMEM((CH,), jnp.int32),
        pltpu.VMEM((CH,), jnp.int32),
        pltpu.VMEM((CH,), jnp.int32),
        pltpu.VMEM((CH,), jnp.int32),
        pltpu.VMEM((CH, DIM // 2), jnp.int32),
        pltpu.VMEM((CH, DIM // 2), jnp.int32),
        pltpu.SemaphoreType.DMA,
        pltpu.SemaphoreType.DMA,
        pltpu.SemaphoreType.DMA,
        pltpu.SemaphoreType.DMA,
    ],
)
def _dispatch(tok_hbm, pos_hbm, xs_hbm,
              pos_all, pos_c0, pos_c1, tok_c0, tok_c1, row0, row1,
              g0, g1, s0, s1):
    wid = lax.axis_index("s") * NC + lax.axis_index("c")
    base = wid * _PW
    pos_c = [pos_c0, pos_c1]
    tok_c = [tok_c0, tok_c1]
    row = [row0, row1]
    gsem = [g0, g1]
    ssem = [s0, s1]
    pltpu.sync_copy(pos_hbm.at[pl.ds(base, _PW)], pos_all)

    def fill(bi, c):
        for j in range(CH // 16):
            sl = pl.ds(16 * j, 16)
            pos_c[bi][sl] = pos_all[pl.ds(c * CH + 16 * j, 16)]
            pvec = jnp.full((16,), base + c * CH + 16 * j, jnp.int32) + \
                lax.iota(jnp.int32, 16)
            tok_c[bi][sl] = ((pvec >> 11) << 10) + (pvec & (TB - 1))

    d_g = [None, None]
    d_s = [None, None]
    fill(0, 0)
    d_g[0] = pltpu.async_copy(tok_hbm.at[tok_c[0]], row[0], gsem[0])
    for c in range(_NCH):
        cb = c % 2
        nb = (c + 1) % 2
        if c + 1 < _NCH:
            if d_s[nb] is not None:
                d_s[nb].wait()
            fill(nb, c + 1)
            d_g[nb] = pltpu.async_copy(tok_hbm.at[tok_c[nb]], row[nb],
                                       gsem[nb])
        d_g[cb].wait()
        d_s[cb] = pltpu.async_copy(row[cb], xs_hbm.at[pos_c[cb]], ssem[cb])
    d_s[0].wait()
    d_s[1].wait()


def _gmm_body(te_ref, xs_ref, f1w_ref, f1b_ref, f2w_ref, f2b_ref, out_ref,
              w1c_ref, w2c_ref, prev_ref):
    t = pl.program_id(0)
    e = te_ref[t]

    @pl.when(jnp.logical_or(t == 0, e != prev_ref[0]))
    def _():
        w1c_ref[...] = f1w_ref[0].astype(jnp.bfloat16)
        w2c_ref[...] = f2w_ref[0].astype(jnp.bfloat16)

    prev_ref[0] = e

    @pl.when(t < te_ref[T_TILES])
    def _():
        xb = _bfunpack_bf16(xs_ref[...])
        h = lax.dot_general(xb, w1c_ref[...],
                            (((1,), (1,)), ((), ())),
                            preferred_element_type=jnp.float32)
        h = h + f1b_ref[0]
        h = 0.5 * h * (1.0 + lax.erf(h * _INV_SQRT2))
        y = lax.dot_general(h.astype(jnp.bfloat16), w2c_ref[...],
                            (((1,), (1,)), ((), ())),
                            preferred_element_type=jnp.float32)
        y = y + f2b_ref[0]
        out_ref[...] = _bfpack(y)


_gmm = pl.pallas_call(
    _gmm_body,
    grid_spec=pltpu.PrefetchScalarGridSpec(
        num_scalar_prefetch=1,
        grid=(T_TILES,),
        in_specs=[
            pl.BlockSpec((BLK, DIM // 2), lambda t, te: (t, 0)),
            pl.BlockSpec((1, HIDDEN, DIM), lambda t, te: (te[t], 0, 0)),
            pl.BlockSpec((1, 1, HIDDEN), lambda t, te: (te[t], 0, 0)),
            pl.BlockSpec((1, DIM, HIDDEN), lambda t, te: (te[t], 0, 0)),
            pl.BlockSpec((1, 1, DIM), lambda t, te: (te[t], 0, 0)),
        ],
        out_specs=pl.BlockSpec((BLK, DIM // 2), lambda t, te: (t, 0)),
        scratch_shapes=[
            pltpu.VMEM((HIDDEN, DIM), jnp.bfloat16),
            pltpu.VMEM((DIM, HIDDEN), jnp.bfloat16),
            pltpu.SMEM((1,), jnp.int32),
        ],
    ),
    out_shape=jax.ShapeDtypeStruct((NROWS, DIM // 2), jnp.int32),
    compiler_params=pltpu.CompilerParams(dimension_semantics=("arbitrary",)),
)


_TW = N_TOK // NWK
_NCHC = _TW // CHC


@functools.partial(
    pl.kernel,
    mesh=_sc_mesh,
    out_type=(
        jax.ShapeDtypeStruct((N_TOK, DIM // 2), jnp.int32),
        jax.ShapeDtypeStruct((N_TOK, DIM // 2), jnp.int32),
    ),
    scratch_types=[
        pltpu.VMEM((_TW,), jnp.int32),
        pltpu.VMEM((_TW,), jnp.int32),
        pltpu.VMEM((CHC,), jnp.int32),
        pltpu.VMEM((CHC,), jnp.int32),
        pltpu.VMEM((CHC,), jnp.int32),
        pltpu.VMEM((CHC,), jnp.int32),
        pltpu.VMEM((CHC, DIM // 2), jnp.int32),
        pltpu.VMEM((CHC, DIM // 2), jnp.int32),
        pltpu.VMEM((CHC, DIM // 2), jnp.int32),
        pltpu.VMEM((CHC, DIM // 2), jnp.int32),
        pltpu.SemaphoreType.DMA,
        pltpu.SemaphoreType.DMA,
        pltpu.SemaphoreType.DMA,
        pltpu.SemaphoreType.DMA,
        pltpu.SemaphoreType.DMA,
        pltpu.SemaphoreType.DMA,
        pltpu.SemaphoreType.DMA,
        pltpu.SemaphoreType.DMA,
    ],
)
def _combine(yw_hbm, pos_hbm, yg0_hbm, yg1_hbm,
             p0_all, p1_all, i00, i01, i10, i11,
             b00, b01, b10, b11,
             g00, g01, g10, g11, s00, s01, s10, s11):
    wid = lax.axis_index("s") * NC + lax.axis_index("c")
    tbase = wid * _TW
    p0base = (tbase // TB) * (2 * TB) + (tbase % TB)
    p1base = p0base + TB
    pltpu.sync_copy(pos_hbm.at[pl.ds(p0base, _TW)], p0_all)
    pltpu.sync_copy(pos_hbm.at[pl.ds(p1base, _TW)], p1_all)

    p_all = [p0_all, p1_all]
    idx = [[i00, i01], [i10, i11]]
    buf = [[b00, b01], [b10, b11]]
    gsem = [[g00, g01], [g10, g11]]
    ssem = [[s00, s01], [s10, s11]]
    out = [yg0_hbm, yg1_hbm]

    def fill(k, bi, c):
        for j in range(CHC // 16):
            sl = pl.ds(16 * j, 16)
            idx[k][bi][sl] = p_all[k][pl.ds(c * CHC + 16 * j, 16)]

    d_g = [[None, None], [None, None]]
    d_s = [[None, None], [None, None]]
    for k in (0, 1):
        fill(k, 0, 0)
        d_g[k][0] = pltpu.async_copy(yw_hbm.at[idx[k][0]], buf[k][0],
                                     gsem[k][0])
    for c in range(_NCHC):
        cb = c % 2
        nb = (c + 1) % 2
        for k in (0, 1):
            if c + 1 < _NCHC:
                if d_s[k][nb] is not None:
                    d_s[k][nb].wait()
                fill(k, nb, c + 1)
                d_g[k][nb] = pltpu.async_copy(yw_hbm.at[idx[k][nb]],
                                              buf[k][nb], gsem[k][nb])
            d_g[k][cb].wait()
            d_s[k][cb] = pltpu.async_copy(
                buf[k][cb],
                out[k].at[pl.ds(tbase + c * CHC, CHC), :], ssem[k][cb])
    for k in (0, 1):
        d_s[k][0].wait()
        d_s[k][1].wait()


def _unpack_f32(p):
    lo = lax.bitcast_convert_type(p << 16, jnp.float32)
    hi = lax.bitcast_convert_type(p & jnp.int32(-65536), jnp.float32)
    return jnp.concatenate([lo, hi], axis=1)


def _finish_body(x_ref, a_ref, b_ref, w0_ref, w1_ref, scale_ref, out_ref):
    s = scale_ref[0, 0]
    a = _unpack_f32(a_ref[...])
    bb = _unpack_f32(b_ref[...])
    out_ref[...] = x_ref[...] + s * (w0_ref[...] * a + w1_ref[...] * bb)


_finish = pl.pallas_call(
    _finish_body,
    grid=(NB,),
    in_specs=[
        pl.BlockSpec((TB, DIM), lambda b: (b, 0)),
        pl.BlockSpec((TB, DIM // 2), lambda b: (b, 0)),
        pl.BlockSpec((TB, DIM // 2), lambda b: (b, 0)),
        pl.BlockSpec((TB, 1), lambda b: (b, 0)),
        pl.BlockSpec((TB, 1), lambda b: (b, 0)),
        pl.BlockSpec((1, 1), lambda b: (0, 0)),
    ],
    out_specs=pl.BlockSpec((TB, DIM), lambda b: (b, 0)),
    out_shape=jax.ShapeDtypeStruct((N_TOK, DIM), jnp.float32),
    compiler_params=pltpu.CompilerParams(dimension_semantics=("arbitrary",)),
)


@jax.jit
def _moe(tokens, router_w, router_b, f1w, f1b, f2w, f2b, scale):
    topi3, rank3, w0, w1, counts_f, tokp = _router(
        tokens, router_w, router_b.reshape(1, E))

    counts = counts_f[:, 0].astype(jnp.int32)
    nt = (counts + BLK - 1) // BLK
    tile_cum = jnp.cumsum(nt)
    start = (BLK * (tile_cum - nt)).astype(jnp.int32)
    t_idx = jnp.arange(T_TILES, dtype=jnp.int32)
    te_map = jnp.minimum(
        jnp.sum((t_idx[:, None] >= tile_cum[None, :]).astype(jnp.int32),
                axis=1), E - 1).astype(jnp.int32)
    te_plus = jnp.concatenate([te_map, tile_cum[-1:]])

    pos3 = _posmap(start, topi3, rank3)
    pos = pos3.reshape(NPAIR)
    xs = _dispatch(tokp, pos)
    yw = _gmm(te_plus, xs,
              f1w, f1b.reshape(E, 1, HIDDEN),
              f2w, f2b.reshape(E, 1, DIM))
    yg0, yg1 = _combine(yw, pos)
    return _finish(tokens, yg0, yg1, w0, w1, scale.reshape(1, 1))


def kernel(x, router_w, router_b, fc1_w, fc1_b, fc2_w, fc2_b, scale):
    b, c, h, w = x.shape
    tokens = jnp.transpose(x, (0, 2, 3, 1)).reshape(b * h * w, c)
    out = _moe(tokens, router_w, router_b, fc1_w, fc1_b, fc2_w, fc2_b,
               scale)
    return jnp.transpose(out.reshape(b, h, w, c), (0, 3, 1, 2))
